# Initial kernel scaffold; baseline (speedup 1.0000x reference)
import jax
import jax.numpy as jnp
from jax.experimental import pallas as pl


def kernel(embedded, idx_pos_map, h0, c0, encoder_outputs, attention_weights, coverage_vec, W_ih, W_hh, b_ih, b_hh, W_enc, b_enc, W_dec, b_dec, W_ei, b_ei, W_pv1, b_pv1, W_pv2, b_pv2):
    raise NotImplementedError("write your pallas kernel here")


# speedup vs baseline: 1.0995x; 1.0995x over previous
import functools

import jax
import jax.numpy as jnp
from jax.experimental import pallas as pl
from jax.experimental.pallas import tpu as pltpu

_C11 = (((1,), (1,)), ((), ()))


def _lstm_body(x_ref, h_ref, c_ref, aw_ref, cov_ref,
               Wih_ref, Whh_ref, bih_ref, bhh_ref,
               h_out_ref, c_out_ref, cov_out_ref):
    H = h_ref.shape[1]
    x = x_ref[...]
    h = h_ref[...]
    c = c_ref[...]
    gates = (jax.lax.dot_general(x, Wih_ref[...], _C11,
                                 preferred_element_type=jnp.float32)
             + bih_ref[...]
             + jax.lax.dot_general(h, Whh_ref[...], _C11,
                                   preferred_element_type=jnp.float32)
             + bhh_ref[...])
    i_g = jax.nn.sigmoid(gates[:, 0 * H:1 * H])
    f_g = jax.nn.sigmoid(gates[:, 1 * H:2 * H])
    g_g = jnp.tanh(gates[:, 2 * H:3 * H])
    o_g = jax.nn.sigmoid(gates[:, 3 * H:4 * H])
    c_new = f_g * c + i_g * g_g
    h_new = o_g * jnp.tanh(c_new)
    h_out_ref[...] = h_new
    c_out_ref[...] = c_new
    cov_out_ref[...] = cov_ref[...] + aw_ref[...]


def _attn_body(s_ref, enc_ref, Wenc_ref, benc_ref, Wdec_ref, bdec_ref,
               Wei_ref, bei_ref,
               at_ref, ctx_ref, e_scr):
    L = enc_ref.shape[0]
    CH = 256
    s_t = s_ref[...]
    s_term = (jax.lax.dot_general(s_t, Wdec_ref[...], _C11,
                                  preferred_element_type=jnp.float32)
              + bdec_ref[...] + benc_ref[...])
    Wenc_bf = Wenc_ref[...].astype(jnp.bfloat16)
    Wei = Wei_ref[...]

    def chunk(j, _):
        rows = enc_ref[pl.ds(j * CH, CH), :].astype(jnp.bfloat16)
        pre = jax.lax.dot_general(rows, Wenc_bf, _C11,
                                  preferred_element_type=jnp.float32)
        pre = jnp.tanh(pre + s_term)
        e_scr[:, pl.ds(j * CH, CH)] = jax.lax.dot_general(
            Wei, pre, _C11, preferred_element_type=jnp.float32)
        return 0

    jax.lax.fori_loop(0, L // CH, chunk, 0)
    e = e_scr[...] + bei_ref[...]
    m = jnp.max(e)
    ex = jnp.exp(e - m)
    a_t = ex / jnp.sum(ex)
    at_ref[...] = a_t
    ctx_ref[...] = jnp.dot(a_t, enc_ref[...],
                           preferred_element_type=jnp.float32)


def _vocab_body(s_ref, ctx_ref, Wpv1_ref, bpv1_ref, bpv2_ref, Wpv2_ref,
                p_ref, a2_scr, a3_scr, *, nb, bv, v):
    H = s_ref.shape[1]
    i = pl.program_id(0)

    @pl.when(i == 0)
    def _():
        a2 = (jax.lax.dot_general(s_ref[...], Wpv1_ref[:, 0:H], _C11,
                                  preferred_element_type=jnp.float32)
              + jax.lax.dot_general(ctx_ref[...], Wpv1_ref[:, H:3 * H], _C11,
                                    preferred_element_type=jnp.float32)
              + bpv1_ref[...])
        a2_scr[...] = a2

    vals = (jax.lax.dot_general(a2_scr[...], Wpv2_ref[...], _C11,
                                preferred_element_type=jnp.float32)
            + bpv2_ref[:, pl.ds(i * bv, bv)])
    a3_scr[:, pl.ds(i * bv, bv)] = vals

    @pl.when(i == nb - 1)
    def _():
        a3 = a3_scr[...]
        mask = jax.lax.broadcasted_iota(jnp.int32, (1, nb * bv), 1) < v
        a3m = jnp.where(mask, a3, -jnp.inf)
        m = jnp.max(a3m)
        ex = jnp.where(mask, jnp.exp(a3m - m), 0.0)
        p_ref[...] = (ex / jnp.sum(ex))[:, 0:v]


def kernel(embedded, idx_pos_map, h0, c0, encoder_outputs, attention_weights,
           coverage_vec, W_ih, W_hh, b_ih, b_hh, W_enc, b_enc, W_dec, b_dec,
           W_ei, b_ei, W_pv1, b_pv1, W_pv2, b_pv2):
    del idx_pos_map
    E = embedded.shape[-1]
    H = h0.shape[-1]
    L = encoder_outputs.shape[0]
    V = W_pv2.shape[0]

    x = embedded.reshape(1, E)
    h = h0.reshape(1, H)
    c = c0.reshape(1, H)
    r2 = lambda b: b.reshape(1, -1)

    h_new, c_new, cov_new = pl.pallas_call(
        _lstm_body,
        out_shape=(
            jax.ShapeDtypeStruct((1, H), jnp.float32),
            jax.ShapeDtypeStruct((1, H), jnp.float32),
            jax.ShapeDtypeStruct((1, L), jnp.float32),
        ),
    )(x, h, c, attention_weights, coverage_vec,
      W_ih, W_hh, r2(b_ih), r2(b_hh))

    a_t, context = pl.pallas_call(
        _attn_body,
        out_shape=(
            jax.ShapeDtypeStruct((1, L), jnp.float32),
            jax.ShapeDtypeStruct((1, 2 * H), jnp.float32),
        ),
        scratch_shapes=[pltpu.VMEM((1, L), jnp.float32)],
    )(h_new, encoder_outputs, W_enc, r2(b_enc), W_dec, r2(b_dec),
      W_ei, r2(b_ei))

    BV = 2048
    NB = pl.cdiv(V, BV)
    b_pv2_pad = jnp.pad(r2(b_pv2), ((0, 0), (0, NB * BV - V)))

    p_vocab = pl.pallas_call(
        functools.partial(_vocab_body, nb=NB, bv=BV, v=V),
        grid=(NB,),
        in_specs=[
            pl.BlockSpec((1, H), lambda i: (0, 0)),
            pl.BlockSpec((1, 2 * H), lambda i: (0, 0)),
            pl.BlockSpec((H, 3 * H), lambda i: (0, 0)),
            pl.BlockSpec((1, H), lambda i: (0, 0)),
            pl.BlockSpec((1, NB * BV), lambda i: (0, 0)),
            pl.BlockSpec((BV, H), lambda i: (i, 0)),
        ],
        out_specs=pl.BlockSpec((1, V), lambda i: (0, 0)),
        out_shape=jax.ShapeDtypeStruct((1, V), jnp.float32),
        scratch_shapes=[
            pltpu.VMEM((1, H), jnp.float32),
            pltpu.VMEM((1, NB * BV), jnp.float32),
        ],
    )(h_new, context, W_pv1, r2(b_pv1), b_pv2_pad, W_pv2)

    return (p_vocab, h_new.reshape(1, 1, H), c_new.reshape(1, 1, H),
            a_t, cov_new)


# speedup vs baseline: 1.1436x; 1.0401x over previous
import functools

import jax
import jax.numpy as jnp
from jax.experimental import pallas as pl
from jax.experimental.pallas import tpu as pltpu

_C11 = (((1,), (1,)), ((), ()))
_F32 = jnp.float32

_NL = 4
_ID = _NL
_IA = _ID + 1
_NA = 8
_IP = _IA + _NA
_NP = 4
_IV = _IP + _NP
_K = 4
_D = 3


def _dot(a, b, dims=_C11):
    return jax.lax.dot_general(a, b, dims, preferred_element_type=_F32)


def _body(x_ref, h0_ref, c0_ref, aw_ref, cov_ref,
          Wih_ref, Whh_ref, bih_ref, bhh_ref,
          enc_ref, Wenc_ref, benc_ref, Wdec_ref, bdec_ref, Wei_ref, bei_ref,
          Wpv1_ref, bpv1_ref, bpv2_ref, Wpv2_hbm,
          p_ref, h_out_ref, c_out_ref, at_ref, cov_out_ref,
          gates_scr, hnew_scr, sterm_scr, wencbf_scr, e_scr, acc_scr,
          a2_scr, a3_scr, ms_scr, vbuf, sems,
          *, H, L, V, nb, bv, last_rows):
    i = pl.program_id(0)
    lch = L // _NA
    pch = H // _NP

    def _start(j):
        slot = jax.lax.rem(j, _K)

        @pl.when(j < nb - 1)
        def _():
            pltpu.make_async_copy(
                Wpv2_hbm.at[pl.ds(j * bv, bv), :],
                vbuf.at[slot], sems.at[slot]).start()

        @pl.when(j == nb - 1)
        def _():
            pltpu.make_async_copy(
                Wpv2_hbm.at[pl.ds(j * bv, last_rows), :],
                vbuf.at[slot, pl.ds(0, last_rows), :],
                sems.at[slot]).start()

    def _wait(j):
        slot = jax.lax.rem(j, _K)

        @pl.when(j < nb - 1)
        def _():
            pltpu.make_async_copy(
                Wpv2_hbm.at[pl.ds(j * bv, bv), :],
                vbuf.at[slot], sems.at[slot]).wait()

        @pl.when(j == nb - 1)
        def _():
            pltpu.make_async_copy(
                Wpv2_hbm.at[pl.ds(j * bv, last_rows), :],
                vbuf.at[slot, pl.ds(0, last_rows), :],
                sems.at[slot]).wait()

    @pl.when(i == 0)
    def _():
        cov_out_ref[...] = cov_ref[...] + aw_ref[...]
        for j in range(_D):
            _start(jnp.int32(j))

    @pl.when(i < _NL)
    def _():
        g = (_dot(x_ref[...], Wih_ref[...]) + _dot(h0_ref[...], Whh_ref[...])
             + bih_ref[:, pl.ds(i * H, H)] + bhh_ref[:, pl.ds(i * H, H)])
        gates_scr[:, pl.ds(i * H, H)] = g

    @pl.when(i == _NL - 1)
    def _():
        g = gates_scr[...]
        i_s = jax.nn.sigmoid(g[:, 0 * H:1 * H])
        f_s = jax.nn.sigmoid(g[:, 1 * H:2 * H])
        g_t = jnp.tanh(g[:, 2 * H:3 * H])
        o_s = jax.nn.sigmoid(g[:, 3 * H:4 * H])
        c_new = f_s * c0_ref[...] + i_s * g_t
        h_new = o_s * jnp.tanh(c_new)
        h_out_ref[...] = h_new
        c_out_ref[...] = c_new
        hnew_scr[...] = h_new

    @pl.when(i == _ID)
    def _():
        sterm_scr[...] = (_dot(hnew_scr[...], Wdec_ref[...])
                          + bdec_ref[...] + benc_ref[...])
        wencbf_scr[...] = Wenc_ref[...].astype(jnp.bfloat16)
        ms_scr[0] = -jnp.inf
        ms_scr[1] = 0.0
        acc_scr[...] = jnp.zeros_like(acc_scr)

    @pl.when(jnp.logical_and(i >= _IA, i < _IA + _NA))
    def _():
        jj = i - _IA
        rows = enc_ref[...]
        pre = _dot(rows.astype(jnp.bfloat16), wencbf_scr[...])
        pre = jnp.tanh(pre + sterm_scr[...])
        e_c = _dot(Wei_ref[...], pre) + bei_ref[...]
        e_scr[:, pl.ds(jj * lch, lch)] = e_c
        m_old = ms_scr[0]
        m_new = jnp.maximum(m_old, jnp.max(e_c))
        corr = jnp.exp(m_old - m_new)
        p = jnp.exp(e_c - m_new)
        acc_scr[...] = (acc_scr[...] * corr
                        + _dot(p, rows, (((1,), (0,)), ((), ()))))
        ms_scr[1] = ms_scr[1] * corr + jnp.sum(p)
        ms_scr[0] = m_new

    @pl.when(i == _IA + _NA - 1)
    def _():
        inv = 1.0 / ms_scr[1]
        at_ref[...] = jnp.exp(e_scr[...] - ms_scr[0]) * inv
        acc_scr[...] = acc_scr[...] * inv

    @pl.when(jnp.logical_and(i >= _IP, i < _IP + _NP))
    def _():
        k = i - _IP
        chunk = Wpv1_ref[...]
        a2_c = (_dot(hnew_scr[...], chunk[:, 0:H])
                + _dot(acc_scr[...], chunk[:, H:3 * H])
                + bpv1_ref[:, pl.ds(k * pch, pch)])
        a2_scr[:, pl.ds(k * pch, pch)] = a2_c

    @pl.when(i >= _IV)
    def _():
        j = i - _IV
        _wait(j)
        vals = (_dot(a2_scr[...], vbuf[jax.lax.rem(j, _K)])
                + bpv2_ref[:, pl.ds(j * bv, bv)])
        a3_scr[:, pl.ds(j * bv, bv)] = vals
        _start(j + _D)

        @pl.when(j == nb - 1)
        def _():
            a3 = a3_scr[...]
            mask = jax.lax.broadcasted_iota(jnp.int32, (1, nb * bv), 1) < V
            a3m = jnp.where(mask, a3, -jnp.inf)
            m = jnp.max(a3m)
            ex = jnp.where(mask, jnp.exp(a3m - m), 0.0)
            p_ref[...] = (ex / jnp.sum(ex))[:, 0:V]


def kernel(embedded, idx_pos_map, h0, c0, encoder_outputs, attention_weights,
           coverage_vec, W_ih, W_hh, b_ih, b_hh, W_enc, b_enc, W_dec, b_dec,
           W_ei, b_ei, W_pv1, b_pv1, W_pv2, b_pv2):
    del idx_pos_map
    E = embedded.shape[-1]
    H = h0.shape[-1]
    L = encoder_outputs.shape[0]
    V = W_pv2.shape[0]

    BV = 1024
    NB = pl.cdiv(V, BV)
    LAST = V - (NB - 1) * BV
    NSTEPS = _IV + NB
    lch = L // _NA
    pch = H // _NP

    x = embedded.reshape(1, E)
    h = h0.reshape(1, H)
    c = c0.reshape(1, H)
    r2 = lambda b: b.reshape(1, -1)
    b_pv2_pad = jnp.pad(r2(b_pv2), ((0, 0), (0, NB * BV - V)))

    const = lambda *_: tuple(0 for _ in range(2))
    specs = [
        pl.BlockSpec((1, E), const),
        pl.BlockSpec((1, H), const),
        pl.BlockSpec((1, H), const),
        pl.BlockSpec((1, L), const),
        pl.BlockSpec((1, L), const),
        pl.BlockSpec((H, E), lambda i: (jnp.minimum(i, _NL - 1), 0)),
        pl.BlockSpec((H, H), lambda i: (jnp.minimum(i, _NL - 1), 0)),
        pl.BlockSpec((1, 4 * H), const),
        pl.BlockSpec((1, 4 * H), const),
        pl.BlockSpec((lch, 2 * H),
                     lambda i: (jnp.clip(i - _IA, 0, _NA - 1), 0)),
        pl.BlockSpec((H, 2 * H), const),
        pl.BlockSpec((1, H), const),
        pl.BlockSpec((H, H), const),
        pl.BlockSpec((1, H), const),
        pl.BlockSpec((1, H), const),
        pl.BlockSpec((1, 1), const),
        pl.BlockSpec((pch, 3 * H),
                     lambda i: (jnp.clip(i - _IP, 0, _NP - 1), 0)),
        pl.BlockSpec((1, H), const),
        pl.BlockSpec((1, NB * BV), const),
        pl.BlockSpec(memory_space=pl.ANY),
    ]

    out_specs = (
        pl.BlockSpec((1, V), const),
        pl.BlockSpec((1, H), const),
        pl.BlockSpec((1, H), const),
        pl.BlockSpec((1, L), const),
        pl.BlockSpec((1, L), const),
    )
    out_shape = (
        jax.ShapeDtypeStruct((1, V), _F32),
        jax.ShapeDtypeStruct((1, H), _F32),
        jax.ShapeDtypeStruct((1, H), _F32),
        jax.ShapeDtypeStruct((1, L), _F32),
        jax.ShapeDtypeStruct((1, L), _F32),
    )

    p_vocab, h_new, c_new, a_t, cov_new = pl.pallas_call(
        functools.partial(_body, H=H, L=L, V=V, nb=NB, bv=BV, last_rows=LAST),
        grid=(NSTEPS,),
        in_specs=specs,
        out_specs=out_specs,
        out_shape=out_shape,
        scratch_shapes=[
            pltpu.VMEM((1, 4 * H), _F32),
            pltpu.VMEM((1, H), _F32),
            pltpu.VMEM((1, H), _F32),
            pltpu.VMEM((H, 2 * H), jnp.bfloat16),
            pltpu.VMEM((1, L), _F32),
            pltpu.VMEM((1, 2 * H), _F32),
            pltpu.VMEM((1, H), _F32),
            pltpu.VMEM((1, NB * BV), _F32),
            pltpu.SMEM((2,), _F32),
            pltpu.VMEM((_K, BV, H), _F32),
            pltpu.SemaphoreType.DMA((_K,)),
        ],
    )(x, h, c, attention_weights, coverage_vec,
      W_ih, W_hh, r2(b_ih), r2(b_hh),
      encoder_outputs, W_enc, r2(b_enc), W_dec, r2(b_dec), W_ei, r2(b_ei),
      W_pv1, r2(b_pv1), b_pv2_pad, W_pv2)

    return (p_vocab, h_new.reshape(1, 1, H), c_new.reshape(1, 1, H),
            a_t, cov_new)
